# initial kernel scaffold (unmeasured)
import jax
import jax.numpy as jnp
from jax import lax
from jax.experimental import pallas as pl
from jax.experimental.pallas import tpu as pltpu

N_DEV = 8
N_EXP = 32
E_PER = N_EXP // N_DEV


def kernel(x, router_W, route_idx, expert_W):
    m, d = x.shape
    e_per, _, h = expert_W.shape

    def body(x_ref, rw_ref, idx_ref, ew_ref, out_ref,
             comm_ref, send_sems, recv_sems):
        my = lax.axis_index("i")
        left = lax.rem(my + N_DEV - 1, N_DEV)
        right = lax.rem(my + 1, N_DEV)

        barrier_sem = pltpu.get_barrier_semaphore()
        pl.semaphore_signal(barrier_sem, inc=1, device_id=(left,),
                            device_id_type=pl.DeviceIdType.MESH)
        pl.semaphore_signal(barrier_sem, inc=1, device_id=(right,),
                            device_id_type=pl.DeviceIdType.MESH)
        pl.semaphore_wait(barrier_sem, 2)

        xv = x_ref[...]
        scores = jnp.dot(xv, rw_ref[...],
                         preferred_element_type=jnp.float32)
        s_max = jnp.max(scores, axis=1, keepdims=True)
        p = jnp.exp(scores - s_max)
        probs = p / jnp.sum(p, axis=1, keepdims=True)
        e0 = idx_ref[:, 0:1]
        e1 = idx_ref[:, 1:2]
        lane = lax.broadcasted_iota(jnp.int32, (m, N_EXP), 1)
        g0 = jnp.sum(jnp.where(lane == e0, probs, 0.0), axis=1, keepdims=True)
        g1 = jnp.sum(jnp.where(lane == e1, probs, 0.0), axis=1, keepdims=True)
        gs = g0 + g1
        g0 = g0 / gs
        g1 = g1 / gs

        def chunk_contrib(owner, w_chunk, acc):
            for j in range(E_PER):
                e = owner * E_PER + j
                coef = (jnp.where(e0 == e, g0, 0.0)
                        + jnp.where(e1 == e, g1, 0.0))
                acc = acc + jnp.dot(xv * coef, w_chunk[j],
                                    preferred_element_type=jnp.float32)
            return acc

        comm_ref[0] = ew_ref[...]
        acc = jnp.zeros((m, h), jnp.float32)

        for hh in range(N_DEV - 1):
            rdma = pltpu.make_async_remote_copy(
                src_ref=comm_ref.at[hh],
                dst_ref=comm_ref.at[hh + 1],
                send_sem=send_sems.at[hh],
                recv_sem=recv_sems.at[hh],
                device_id=(right,),
                device_id_type=pl.DeviceIdType.MESH,
            )
            rdma.start()
            owner = lax.rem(my + N_DEV - hh, N_DEV) if hh else my
            acc = chunk_contrib(owner, comm_ref[hh], acc)
            rdma.wait()

        owner = lax.rem(my + 1, N_DEV)
        acc = chunk_contrib(owner, comm_ref[N_DEV - 1], acc)

        out_ref[...] = acc

    return pl.pallas_call(
        body,
        out_shape=jax.ShapeDtypeStruct((m, h), jnp.float32),
        in_specs=[pl.BlockSpec(memory_space=pltpu.VMEM)] * 4,
        out_specs=pl.BlockSpec(memory_space=pltpu.VMEM),
        scratch_shapes=[
            pltpu.VMEM((N_DEV, e_per, d, h), jnp.float32),
            pltpu.SemaphoreType.DMA((N_DEV - 1,)),
            pltpu.SemaphoreType.DMA((N_DEV - 1,)),
        ],
        compiler_params=pltpu.CompilerParams(collective_id=0),
    )(x, router_W, route_idx, expert_W)


# baseline (device time: 183160 ns/iter reference)
import jax
import jax.numpy as jnp
from jax import lax
from jax.experimental import pallas as pl
from jax.experimental.pallas import tpu as pltpu

N_DEV = 8
N_EXP = 32
E_PER = N_EXP // N_DEV


def kernel(x, router_W, route_idx, expert_W):
    m, d = x.shape
    e_per, _, h = expert_W.shape

    def body(x_ref, rw_ref, idx_ref, ew_ref, out_ref,
             comm_ref, send_sems, recv_sems):
        my = lax.axis_index("i")
        left = lax.rem(my + N_DEV - 1, N_DEV)
        right = lax.rem(my + 1, N_DEV)

        barrier_sem = pltpu.get_barrier_semaphore()
        pl.semaphore_signal(barrier_sem, inc=1, device_id=(left,),
                            device_id_type=pl.DeviceIdType.MESH)
        pl.semaphore_signal(barrier_sem, inc=1, device_id=(right,),
                            device_id_type=pl.DeviceIdType.MESH)
        pl.semaphore_wait(barrier_sem, 2)

        xv = x_ref[...]
        scores = jnp.dot(xv, rw_ref[...],
                         preferred_element_type=jnp.float32)
        s_max = jnp.max(scores, axis=1, keepdims=True)
        p = jnp.exp(scores - s_max)
        probs = p / jnp.sum(p, axis=1, keepdims=True)
        e0 = idx_ref[:, 0:1]
        e1 = idx_ref[:, 1:2]
        lane = lax.broadcasted_iota(jnp.int32, (m, N_EXP), 1)
        g0 = jnp.sum(jnp.where(lane == e0, probs, 0.0), axis=1, keepdims=True)
        g1 = jnp.sum(jnp.where(lane == e1, probs, 0.0), axis=1, keepdims=True)
        gs = g0 + g1
        g0 = g0 / gs
        g1 = g1 / gs

        def chunk_contrib(owner, w_chunk, acc):
            for j in range(E_PER):
                e = owner * E_PER + j
                coef = (jnp.where(e0 == e, g0, 0.0)
                        + jnp.where(e1 == e, g1, 0.0))
                acc = acc + jnp.dot(xv * coef, w_chunk[j],
                                    preferred_element_type=jnp.float32)
            return acc

        comm_ref[0] = ew_ref[...]
        acc = jnp.zeros((m, h), jnp.float32)

        for hh in range(N_DEV - 1):
            rdma = pltpu.make_async_remote_copy(
                src_ref=comm_ref.at[hh],
                dst_ref=comm_ref.at[hh + 1],
                send_sem=send_sems.at[hh],
                recv_sem=recv_sems.at[hh],
                device_id=(right,),
                device_id_type=pl.DeviceIdType.MESH,
            )
            rdma.start()
            owner = lax.rem(my + N_DEV - hh, N_DEV)
            acc = chunk_contrib(owner, comm_ref[hh], acc)
            rdma.wait()

        owner = lax.rem(my + 1, N_DEV)
        acc = chunk_contrib(owner, comm_ref[N_DEV - 1], acc)

        out_ref[...] = acc

    return pl.pallas_call(
        body,
        out_shape=jax.ShapeDtypeStruct((m, h), jnp.float32),
        in_specs=[pl.BlockSpec(memory_space=pltpu.VMEM)] * 4,
        out_specs=pl.BlockSpec(memory_space=pltpu.VMEM),
        scratch_shapes=[
            pltpu.VMEM((N_DEV, e_per, d, h), jnp.float32),
            pltpu.SemaphoreType.DMA((N_DEV - 1,)),
            pltpu.SemaphoreType.DMA((N_DEV - 1,)),
        ],
        compiler_params=pltpu.CompilerParams(collective_id=0),
    )(x, router_W, route_idx, expert_W)


# device time: 111968 ns/iter; 1.6358x vs baseline; 1.6358x over previous
import jax
import jax.numpy as jnp
from jax import lax
from jax.experimental import pallas as pl
from jax.experimental.pallas import tpu as pltpu

N_DEV = 8
N_EXP = 32
E_PER = N_EXP // N_DEV


def kernel(x, router_W, route_idx, expert_W):
    m, d = x.shape
    e_per, _, h = expert_W.shape

    def body(x_ref, rw_ref, idx_ref, ew_ref, out_ref,
             comm_ref, send_sems, recv_sems):
        my = lax.axis_index("i")
        left = lax.rem(my + N_DEV - 1, N_DEV)
        right = lax.rem(my + 1, N_DEV)

        barrier_sem = pltpu.get_barrier_semaphore()
        pl.semaphore_signal(barrier_sem, inc=1, device_id=(left,),
                            device_id_type=pl.DeviceIdType.MESH)
        pl.semaphore_signal(barrier_sem, inc=1, device_id=(right,),
                            device_id_type=pl.DeviceIdType.MESH)
        pl.semaphore_wait(barrier_sem, 2)

        xv = x_ref[...]
        scores = jnp.dot(xv, rw_ref[...],
                         preferred_element_type=jnp.float32)
        s_max = jnp.max(scores, axis=1, keepdims=True)
        p = jnp.exp(scores - s_max)
        probs = p / jnp.sum(p, axis=1, keepdims=True)
        e0 = idx_ref[:, 0:1]
        e1 = idx_ref[:, 1:2]
        lane = lax.broadcasted_iota(jnp.int32, (m, N_EXP), 1)
        g0 = jnp.sum(jnp.where(lane == e0, probs, 0.0), axis=1, keepdims=True)
        g1 = jnp.sum(jnp.where(lane == e1, probs, 0.0), axis=1, keepdims=True)
        gs = g0 + g1
        g0 = g0 / gs
        g1 = g1 / gs

        def contrib(origin_offset, slot, acc):
            owner = lax.rem(my + N_DEV + origin_offset, N_DEV)
            w_chunk = comm_ref[slot]
            for j in range(E_PER):
                e = owner * E_PER + j
                coef = (jnp.where(e0 == e, g0, 0.0)
                        + jnp.where(e1 == e, g1, 0.0))
                acc = acc + jnp.dot(xv * coef, w_chunk[j],
                                    preferred_element_type=jnp.float32)
            return acc

        def send(src_slot, dst_slot, sem_idx, target):
            return pltpu.make_async_remote_copy(
                src_ref=comm_ref.at[src_slot],
                dst_ref=comm_ref.at[dst_slot],
                send_sem=send_sems.at[sem_idx],
                recv_sem=recv_sems.at[sem_idx],
                device_id=(target,),
                device_id_type=pl.DeviceIdType.MESH,
            )

        comm_ref[0] = ew_ref[...]
        acc = jnp.zeros((m, h), jnp.float32)

        l_src = [0, 5, 6]
        for k in range(4):
            r = send(k, k + 1, k, right)
            r.start()
            lft = None
            if k < 3:
                lft = send(l_src[k], 5 + k, 4 + k, left)
                lft.start()
            if k == 0:
                acc = contrib(0, 0, acc)
            else:
                acc = contrib(-k, k, acc)
                acc = contrib(k, 4 + k, acc)
            r.wait()
            if lft is not None:
                lft.wait()
        acc = contrib(-4, 4, acc)

        out_ref[...] = acc

    return pl.pallas_call(
        body,
        out_shape=jax.ShapeDtypeStruct((m, h), jnp.float32),
        in_specs=[pl.BlockSpec(memory_space=pltpu.VMEM)] * 4,
        out_specs=pl.BlockSpec(memory_space=pltpu.VMEM),
        scratch_shapes=[
            pltpu.VMEM((N_DEV, e_per, d, h), jnp.float32),
            pltpu.SemaphoreType.DMA((7,)),
            pltpu.SemaphoreType.DMA((7,)),
        ],
        compiler_params=pltpu.CompilerParams(collective_id=0),
    )(x, router_W, route_idx, expert_W)


# device time: 40787 ns/iter; 4.4906x vs baseline; 2.7452x over previous
import jax
import jax.numpy as jnp
from jax import lax
from jax.experimental import pallas as pl
from jax.experimental.pallas import tpu as pltpu

N_DEV = 8
N_EXP = 32
E_PER = N_EXP // N_DEV

DIMS = (1, 3, 4)
PERMS = ((1, 3, 4), (3, 4, 1), (4, 1, 3))
FRACS = ((0, 80), (80, 80), (160, 96))
A, B0, B1, C0, C1, C2, C3 = range(7)


def kernel(x, router_W, route_idx, expert_W):
    m, d = x.shape
    e_per, _, h = expert_W.shape

    def body(x_ref, rw_ref, idx_ref, ew_ref, out_ref,
             comm_ref, ew16_ref, send_sems, recv_sems):
        my = lax.axis_index("i")

        barrier_sem = pltpu.get_barrier_semaphore()
        for dd in DIMS:
            pl.semaphore_signal(barrier_sem, inc=1,
                                device_id=(jnp.bitwise_xor(my, dd),),
                                device_id_type=pl.DeviceIdType.MESH)
        pl.semaphore_wait(barrier_sem, 3)

        def mk(p, t, mask, link):
            off, sz = FRACS[p]
            fsl = pl.ds(off, sz)
            src = (ew16_ref.at[:, fsl, :] if mask == 0
                   else comm_ref.at[mask, :, fsl, :])
            return pltpu.make_async_remote_copy(
                src_ref=src,
                dst_ref=comm_ref.at[mask ^ link, :, fsl, :],
                send_sem=send_sems.at[p * 7 + t],
                recv_sem=recv_sems.at[p * 7 + t],
                device_id=(jnp.bitwise_xor(my, link),),
                device_id_type=pl.DeviceIdType.MESH,
            )

        ew16_ref[...] = ew_ref[...].astype(jnp.bfloat16)

        a_ = [mk(p, A, 0, PERMS[p][0]) for p in range(3)]
        b0 = [mk(p, B0, 0, PERMS[p][1]) for p in range(3)]
        b1 = [mk(p, B1, PERMS[p][0], PERMS[p][1]) for p in range(3)]
        c0 = [mk(p, C0, 0, PERMS[p][2]) for p in range(3)]
        c1 = [mk(p, C1, PERMS[p][0], PERMS[p][2]) for p in range(3)]
        c2 = [mk(p, C2, PERMS[p][1], PERMS[p][2]) for p in range(3)]
        c3 = [mk(p, C3, PERMS[p][0] ^ PERMS[p][1], PERMS[p][2])
              for p in range(3)]

        for r in a_:
            r.start()
        for r in b0:
            r.start()
        for r in c0:
            r.start()

        xv = x_ref[...]
        scores = jnp.dot(xv, rw_ref[...],
                         preferred_element_type=jnp.float32)
        s_max = jnp.max(scores, axis=1, keepdims=True)
        p_ = jnp.exp(scores - s_max)
        probs = p_ / jnp.sum(p_, axis=1, keepdims=True)
        e0 = idx_ref[:, 0:1]
        e1 = idx_ref[:, 1:2]
        lane = lax.broadcasted_iota(jnp.int32, (m, N_EXP), 1)
        g0 = jnp.sum(jnp.where(lane == e0, probs, 0.0), axis=1, keepdims=True)
        g1 = jnp.sum(jnp.where(lane == e1, probs, 0.0), axis=1, keepdims=True)
        gs = g0 + g1
        g0 = g0 / gs
        g1 = g1 / gs

        def contrib(mask, w_chunk, acc):
            owner = jnp.bitwise_xor(my, mask)
            for j in range(E_PER):
                e = owner * E_PER + j
                coef = (jnp.where(e0 == e, g0, 0.0)
                        + jnp.where(e1 == e, g1, 0.0))
                xs = (xv * coef).astype(jnp.bfloat16)
                acc = acc + jnp.dot(xs, w_chunk[j],
                                    preferred_element_type=jnp.float32)
            return acc

        acc = jnp.zeros((m, h), jnp.float32)
        acc = contrib(0, ew16_ref[...], acc)

        for p in range(3):
            a_[p].wait()
            b1[p].start()
            c1[p].start()
        for p in range(3):
            b0[p].wait()
            c2[p].start()
        for p in range(3):
            b1[p].wait()
            c3[p].start()

        for p in range(3):
            c0[p].wait()
        for mask in DIMS:
            acc = contrib(mask, comm_ref[mask], acc)

        for p in range(3):
            c1[p].wait()
        for p in range(3):
            c2[p].wait()
        for mask in (2, 5, 7):
            acc = contrib(mask, comm_ref[mask], acc)

        for p in range(3):
            c3[p].wait()
        acc = contrib(6, comm_ref[6], acc)

        out_ref[...] = acc

    return pl.pallas_call(
        body,
        out_shape=jax.ShapeDtypeStruct((m, h), jnp.float32),
        in_specs=[pl.BlockSpec(memory_space=pltpu.VMEM)] * 4,
        out_specs=pl.BlockSpec(memory_space=pltpu.VMEM),
        scratch_shapes=[
            pltpu.VMEM((N_DEV, e_per, d, h), jnp.bfloat16),
            pltpu.VMEM((e_per, d, h), jnp.bfloat16),
            pltpu.SemaphoreType.DMA((21,)),
            pltpu.SemaphoreType.DMA((21,)),
        ],
        compiler_params=pltpu.CompilerParams(collective_id=0),
    )(x, router_W, route_idx, expert_W)
